# baseline (device time: 287037 ns/iter reference)
import os

import jax

jax.config.update("jax_compilation_cache_dir", "/tmp/jax_persist_cache")
jax.config.update("jax_persistent_cache_min_entry_size_bytes", -1)
jax.config.update("jax_persistent_cache_min_compile_time_secs", 0)

import jax.numpy as jnp
from jax import lax
from jax.experimental import pallas as pl
from jax.experimental.pallas import tpu as pltpu

N_DEV = 16
SQ = 2048
SKV = 2048
HQ_LOCAL = 8
DH = 128
D_MODEL = 1024
SCALE = 0.08838834764831843
CHUNK = SQ // N_DEV
HALF = D_MODEL // 2
ROW_BLK = 512
N_RB = SQ // ROW_BLK
WIN = ROW_BLK + 2 * 128
NEG = -1e9


def kernel(x, Wq, K_ext, V_ext, Wo):
    idx = lax.axis_index("i")
    x2 = x[0]
    K = lax.dynamic_slice(K_ext, (0, 0, idx * HQ_LOCAL, 0), (1, SKV, HQ_LOCAL, DH))[0]
    V = lax.dynamic_slice(V_ext, (0, 0, idx * HQ_LOCAL, 0), (1, SKV, HQ_LOCAL, DH))[0]
    K = jnp.transpose(K, (1, 0, 2))
    V = jnp.transpose(V, (1, 0, 2))

    def body(x_ref, wq_ref, k_ref, v_ref, wo_ref, out_ref,
             comm_f, comm_r, send_f, recv_f, send_r, recv_r):
        my = lax.axis_index("i")
        right = lax.rem(my + 1, N_DEV)
        left = lax.rem(my + N_DEV - 1, N_DEV)

        q = jnp.dot(x_ref[:, :], wq_ref[:, :], preferred_element_type=jnp.float32)

        for r in range(N_RB):
            r0 = r * ROW_BLK
            acc = jnp.zeros((ROW_BLK, D_MODEL), jnp.float32)
            if r == 0:
                qi = lax.broadcasted_iota(jnp.int32, (ROW_BLK, SKV), 0)
                ki = lax.broadcasted_iota(jnp.int32, (ROW_BLK, SKV), 1)
                mask = (jnp.abs(qi - ki) <= 128) | (ki < 32) | (qi < 32)
                bias = jnp.where(mask, 0.0, NEG).astype(jnp.float32)
                for h in range(HQ_LOCAL):
                    qh = q[r0:r0 + ROW_BLK, h * DH:(h + 1) * DH]
                    s = lax.dot_general(
                        qh, k_ref[h], (((1,), (1,)), ((), ())),
                        preferred_element_type=jnp.float32) * SCALE + bias
                    m = jnp.max(s, axis=1, keepdims=True)
                    w = jnp.exp(s - m)
                    denom = jnp.sum(w, axis=1, keepdims=True)
                    ctxh = jnp.dot(w, v_ref[h],
                                   preferred_element_type=jnp.float32) / denom
                    acc = acc + jnp.dot(ctxh, wo_ref[h * DH:(h + 1) * DH, :],
                                        preferred_element_type=jnp.float32)
            else:
                c0 = r0 - 128
                win = min(c0 + WIN, SKV) - c0
                qi_w = lax.broadcasted_iota(jnp.int32, (ROW_BLK, win), 0) + r0
                ki_w = lax.broadcasted_iota(jnp.int32, (ROW_BLK, win), 1) + c0
                bias_w = jnp.where(jnp.abs(qi_w - ki_w) <= 128, 0.0, NEG)
                bias_w = bias_w.astype(jnp.float32)
                ki_g = lax.broadcasted_iota(jnp.int32, (ROW_BLK, 128), 1)
                bias_g = jnp.where(ki_g < 32, 0.0, NEG).astype(jnp.float32)
                for h in range(HQ_LOCAL):
                    qh = q[r0:r0 + ROW_BLK, h * DH:(h + 1) * DH]
                    s_w = lax.dot_general(
                        qh, k_ref[h, c0:c0 + win, :], (((1,), (1,)), ((), ())),
                        preferred_element_type=jnp.float32) * SCALE + bias_w
                    s_g = lax.dot_general(
                        qh, k_ref[h, 0:128, :], (((1,), (1,)), ((), ())),
                        preferred_element_type=jnp.float32) * SCALE + bias_g
                    m = jnp.maximum(jnp.max(s_w, axis=1, keepdims=True),
                                    jnp.max(s_g, axis=1, keepdims=True))
                    w_w = jnp.exp(s_w - m)
                    w_g = jnp.exp(s_g - m)
                    denom = (jnp.sum(w_w, axis=1, keepdims=True)
                             + jnp.sum(w_g, axis=1, keepdims=True))
                    ctxh = (jnp.dot(w_w, v_ref[h, c0:c0 + win, :],
                                    preferred_element_type=jnp.float32)
                            + jnp.dot(w_g, v_ref[h, 0:128, :],
                                      preferred_element_type=jnp.float32)) / denom
                    acc = acc + jnp.dot(ctxh, wo_ref[h * DH:(h + 1) * DH, :],
                                        preferred_element_type=jnp.float32)
            out_ref[r0:r0 + ROW_BLK, :] = acc

        if os.environ.get("NO_RING") == "1":
            return

        for s_ in range(N_DEV - 1):
            slot = s_ % 2
            f_send = lax.rem(my - s_ + 2 * N_DEV, N_DEV)
            f_recv = lax.rem(my - s_ - 1 + 2 * N_DEV, N_DEV)
            r_send = lax.rem(my + s_, N_DEV)
            r_recv = lax.rem(my + s_ + 1, N_DEV)
            rf = pltpu.make_async_remote_copy(
                src_ref=out_ref.at[pl.ds(f_send * CHUNK, CHUNK), pl.ds(0, HALF)],
                dst_ref=comm_f.at[slot],
                send_sem=send_f.at[slot],
                recv_sem=recv_f.at[slot],
                device_id=(right,),
                device_id_type=pl.DeviceIdType.MESH,
            )
            rr = pltpu.make_async_remote_copy(
                src_ref=out_ref.at[pl.ds(r_send * CHUNK, CHUNK), pl.ds(HALF, HALF)],
                dst_ref=comm_r.at[slot],
                send_sem=send_r.at[slot],
                recv_sem=recv_r.at[slot],
                device_id=(left,),
                device_id_type=pl.DeviceIdType.MESH,
            )
            rf.start()
            rr.start()
            rf.wait()
            out_ref[pl.ds(f_recv * CHUNK, CHUNK), 0:HALF] = (
                out_ref[pl.ds(f_recv * CHUNK, CHUNK), 0:HALF] + comm_f[slot])
            rr.wait()
            out_ref[pl.ds(r_recv * CHUNK, CHUNK), HALF:D_MODEL] = (
                out_ref[pl.ds(r_recv * CHUNK, CHUNK), HALF:D_MODEL] + comm_r[slot])

        for s_ in range(N_DEV - 1):
            slot = (N_DEV - 1 + s_) % 2
            f_c = lax.rem(my + 1 - s_ + 2 * N_DEV, N_DEV)
            r_c = lax.rem(my - 1 + s_ + 2 * N_DEV, N_DEV)
            rf = pltpu.make_async_remote_copy(
                src_ref=out_ref.at[pl.ds(f_c * CHUNK, CHUNK), pl.ds(0, HALF)],
                dst_ref=out_ref.at[pl.ds(f_c * CHUNK, CHUNK), pl.ds(0, HALF)],
                send_sem=send_f.at[slot],
                recv_sem=recv_f.at[slot],
                device_id=(right,),
                device_id_type=pl.DeviceIdType.MESH,
            )
            rr = pltpu.make_async_remote_copy(
                src_ref=out_ref.at[pl.ds(r_c * CHUNK, CHUNK), pl.ds(HALF, HALF)],
                dst_ref=out_ref.at[pl.ds(r_c * CHUNK, CHUNK), pl.ds(HALF, HALF)],
                send_sem=send_r.at[slot],
                recv_sem=recv_r.at[slot],
                device_id=(left,),
                device_id_type=pl.DeviceIdType.MESH,
            )
            rf.start()
            rr.start()
            rf.wait()
            rr.wait()

    out = pl.pallas_call(
        body,
        out_shape=jax.ShapeDtypeStruct((SQ, D_MODEL), jnp.float32),
        in_specs=[pl.BlockSpec(memory_space=pltpu.VMEM)] * 5,
        out_specs=pl.BlockSpec(memory_space=pltpu.VMEM),
        scratch_shapes=[
            pltpu.VMEM((2, CHUNK, HALF), jnp.float32),
            pltpu.VMEM((2, CHUNK, HALF), jnp.float32),
            pltpu.SemaphoreType.DMA((2,)),
            pltpu.SemaphoreType.DMA((2,)),
            pltpu.SemaphoreType.DMA((2,)),
            pltpu.SemaphoreType.DMA((2,)),
        ],
        compiler_params=pltpu.CompilerParams(
            vmem_limit_bytes=100 * 1024 * 1024,
        ),
    )(x2, Wq, K, V, Wo)
    return out[None]
